# baseline (speedup 1.0000x reference)
import functools

import jax
import jax.numpy as jnp
from jax import lax
from jax.experimental import pallas as pl
from jax.experimental.pallas import tpu as pltpu
from jax.experimental.pallas import tpu_sc as plsc

_N_NODES = 100000
_N_REGIONS = 8192
_N_BOUNDARY = 50000
_B = 64

_NW = 32
_CH = 80
_NCHUNK = _N_BOUNDARY // _CH
_ROUNDS = -(-_NCHUNK // _NW)

_RBLK = 4096
_NRBLK = _N_REGIONS // _RBLK


@functools.lru_cache(maxsize=1)
def _build_gather():
    mesh = plsc.VectorSubcoreMesh(core_axis_name="c", subcore_axis_name="s")

    @functools.partial(
        pl.kernel,
        mesh=mesh,
        out_type=(
            jax.ShapeDtypeStruct((_NW, _B, 128), jnp.float32),
            jax.ShapeDtypeStruct((_NW * _B,), jnp.float32),
        ),
        scratch_types=[
            pltpu.VMEM((_CH,), jnp.int32),
            pltpu.VMEM((_CH,), jnp.int32),
            pltpu.VMEM((_CH,), jnp.int32),
            pltpu.VMEM((_CH,), jnp.float32),
            pltpu.VMEM((_CH, 128), jnp.float32),
            pltpu.VMEM((_B, 128), jnp.float32),
            pltpu.VMEM((_B,), jnp.float32),
            pltpu.VMEM_SHARED((16 * _B, 128), jnp.float32),
            pltpu.VMEM_SHARED((16 * _B,), jnp.float32),
            pltpu.SemaphoreType.DMA,
            pltpu.SemaphoreType.DMA,
        ],
    )
    def gather_kernel(bn_hbm, nbi_hbm, emb_hbm, z2_hbm, z1_hbm,
                      sum_out, cnt_out,
                      idx_v, bbi_v, bbi2_v, ones_v, rows_v, osum_v, ocnt_v,
                      acc_sh, cnt_sh, sem1, sem2):
        c = lax.axis_index("c")
        s = lax.axis_index("s")
        wid = c * 16 + s

        for i in range(_CH // 16):
            ones_v[pl.ds(i * 16, 16)] = jnp.ones((16,), jnp.float32)

        @pl.when(s == 0)
        def _():
            pltpu.sync_copy(z2_hbm, acc_sh)
            pltpu.sync_copy(z1_hbm, cnt_sh)

        plsc.subcore_barrier()

        for k in range(_ROUNDS):
            cid = wid + _NW * k

            @pl.when(cid < _NCHUNK)
            def _():
                pltpu.sync_copy(bn_hbm.at[pl.ds(cid * _CH, _CH)], idx_v)
                cp1 = pltpu.async_copy(nbi_hbm.at[idx_v], bbi_v, sem1)
                cp2 = pltpu.async_copy(emb_hbm.at[idx_v], rows_v, sem2)
                cp1.wait()
                cp2.wait()
                for i in range(_CH // 16):
                    bbi2_v[pl.ds(i * 16, 16)] = (
                        bbi_v[pl.ds(i * 16, 16)] + s * _B)
                pltpu.sync_copy(rows_v, acc_sh.at[bbi2_v], add=True)
                pltpu.sync_copy(ones_v, cnt_sh.at[bbi2_v], add=True)

        plsc.subcore_barrier()
        pltpu.sync_copy(acc_sh.at[pl.ds(s * _B, _B)], osum_v)
        pltpu.sync_copy(cnt_sh.at[pl.ds(s * _B, _B)], ocnt_v)
        pltpu.sync_copy(osum_v, sum_out.at[wid])
        pltpu.sync_copy(ocnt_v, cnt_out.at[pl.ds(wid * _B, _B)])

    return gather_kernel


def _region_body(rbi_ref, reg_ref, accr_ref, cntr_ref):
    i = pl.program_id(0)

    @pl.when(i == 0)
    def _():
        accr_ref[:, :] = jnp.zeros_like(accr_ref)
        cntr_ref[:, :] = jnp.zeros_like(cntr_ref)

    rbi = rbi_ref[0]
    segr = lax.broadcasted_iota(jnp.int32, (_B, _RBLK), 0)
    mr = (segr == rbi).astype(jnp.float32)
    accr_ref[:, :] += jnp.dot(mr, reg_ref[:, :],
                              preferred_element_type=jnp.float32)
    cntr_ref[:, :] += jnp.sum(mr, axis=1, keepdims=True)


def _combine_body(sums_ref, cnts_ref, accr_ref, cntr_ref,
                  ws1_ref, bs1_ref, ws2_ref, bs2_ref, wb1_ref, bb1_ref,
                  wb2_ref, bb2_ref, wv1a_ref, wv1b_ref, bv1_ref,
                  wv2_ref, bv2_ref, out_ref):
    acc_n = sums_ref[pl.ds(0, _B), :]
    for k in range(1, _NW):
        acc_n = acc_n + sums_ref[pl.ds(k * _B, _B), :]
    cnt_row = jnp.sum(cnts_ref[:, :], axis=0, keepdims=True)
    eye = (lax.broadcasted_iota(jnp.int32, (_B, _B), 0)
           == lax.broadcasted_iota(jnp.int32, (_B, _B), 1))
    cnt_n = jnp.sum(jnp.where(eye, cnt_row, 0.0), axis=1, keepdims=True)

    region_mean = accr_ref[:, :] / jnp.maximum(cntr_ref[:, :], 1.0)
    h = jnp.maximum(
        jnp.dot(region_mean, ws1_ref[:, :], preferred_element_type=jnp.float32)
        + bs1_ref[:, :], 0.0)
    gs = jnp.maximum(
        jnp.dot(h, ws2_ref[:, :], preferred_element_type=jnp.float32)
        + bs2_ref[:, :], 0.0)

    bmean = acc_n / jnp.maximum(cnt_n, 1.0)
    hb = jnp.maximum(
        jnp.dot(bmean, wb1_ref[:, :], preferred_element_type=jnp.float32)
        + bb1_ref[:, :], 0.0)
    binfo = (jnp.dot(hb, wb2_ref[:, :], preferred_element_type=jnp.float32)
             + bb2_ref[:, :])

    hv = jnp.maximum(
        jnp.dot(gs, wv1a_ref[:, :], preferred_element_type=jnp.float32)
        + jnp.dot(binfo, wv1b_ref[:, :], preferred_element_type=jnp.float32)
        + bv1_ref[:, :], 0.0)
    out_ref[:, :] = (jnp.dot(hv, wv2_ref[:, :],
                             preferred_element_type=jnp.float32)
                     + bv2_ref[:, :])


def _full_spec(arr):
    nd = arr.ndim
    return pl.BlockSpec(arr.shape, lambda i: (0,) * nd)


def kernel(node_embeddings, region_embeddings, boundary_nodes,
           node_batch_idx, region_batch_idx, action_mask,
           W_s1, b_s1, W_s2, b_s2, W_b1, b_b1, W_b2, b_b2,
           W_v1, b_v1, W_v2, b_v2):
    del action_mask

    sums, cnts = _build_gather()(
        boundary_nodes.astype(jnp.int32),
        node_batch_idx.astype(jnp.int32),
        node_embeddings,
        jnp.zeros((16 * _B, 128), jnp.float32),
        jnp.zeros((16 * _B,), jnp.float32),
    )

    rbi3 = region_batch_idx.astype(jnp.int32).reshape(_NRBLK, 1, _RBLK)
    accr, cntr = pl.pallas_call(
        _region_body,
        grid=(_NRBLK,),
        in_specs=[
            pl.BlockSpec((1, 1, _RBLK), lambda i: (i, 0, 0)),
            pl.BlockSpec((_RBLK, 128), lambda i: (i, 0)),
        ],
        out_specs=[
            pl.BlockSpec((_B, 128), lambda i: (0, 0)),
            pl.BlockSpec((_B, 1), lambda i: (0, 0)),
        ],
        out_shape=[
            jax.ShapeDtypeStruct((_B, 128), jnp.float32),
            jax.ShapeDtypeStruct((_B, 1), jnp.float32),
        ],
    )(rbi3, region_embeddings)

    wv1a = W_v1[:128]
    wv1b = W_v1[128:]
    weights = (W_s1, b_s1.reshape(1, -1), W_s2, b_s2.reshape(1, -1),
               W_b1, b_b1.reshape(1, -1), W_b2, b_b2.reshape(1, -1),
               wv1a, wv1b, b_v1.reshape(1, -1), W_v2, b_v2.reshape(1, -1))

    sums2 = sums.reshape(_NW * _B, 128)
    cnts2 = cnts.reshape(_NW, _B)

    out = pl.pallas_call(
        _combine_body,
        grid=(1,),
        in_specs=[
            pl.BlockSpec((_NW * _B, 128), lambda i: (0, 0)),
            pl.BlockSpec((_NW, _B), lambda i: (0, 0)),
            pl.BlockSpec((_B, 128), lambda i: (0, 0)),
            pl.BlockSpec((_B, 1), lambda i: (0, 0)),
        ] + [_full_spec(w) for w in weights],
        out_specs=pl.BlockSpec((_B, 1), lambda i: (0, 0)),
        out_shape=jax.ShapeDtypeStruct((_B, 1), jnp.float32),
    )(sums2, cnts2, accr, cntr, *weights)
    return out[:, 0]


# speedup vs baseline: 1.4625x; 1.4625x over previous
import functools

import jax
import jax.numpy as jnp
from jax import lax
from jax.experimental import pallas as pl
from jax.experimental.pallas import tpu as pltpu
from jax.experimental.pallas import tpu_sc as plsc

_N_NODES = 100000
_N_REGIONS = 8192
_N_BOUNDARY = 50000
_B = 64

_NW = 32
_PER_TILE = 1560
_CH = 120
_ROUNDS = _PER_TILE // _CH
_EXTRA = _N_BOUNDARY - _NW * _PER_TILE


def _lane_offsets(n):
    offs = list(range(0, n - 15, 16))
    if n % 16:
        offs.append(n - 16)
    return offs

_RBLK = 4096
_NRBLK = _N_REGIONS // _RBLK


@functools.lru_cache(maxsize=1)
def _build_gather():
    mesh = plsc.VectorSubcoreMesh(core_axis_name="c", subcore_axis_name="s")

    @functools.partial(
        pl.kernel,
        mesh=mesh,
        out_type=(
            jax.ShapeDtypeStruct((_NW, _B, 128), jnp.float32),
            jax.ShapeDtypeStruct((_NW * _B,), jnp.float32),
        ),
        scratch_types=[
            pltpu.VMEM((_PER_TILE,), jnp.int32),
            pltpu.VMEM((_CH,), jnp.int32),
            pltpu.VMEM((_CH,), jnp.int32),
            pltpu.VMEM((_CH,), jnp.int32),
            pltpu.VMEM((_CH,), jnp.int32),
            pltpu.VMEM((_CH,), jnp.int32),
            pltpu.VMEM((_CH,), jnp.int32),
            pltpu.VMEM((_CH,), jnp.float32),
            pltpu.VMEM((_CH, 128), jnp.float32),
            pltpu.VMEM((_CH, 128), jnp.float32),
            pltpu.VMEM((_EXTRA,), jnp.int32),
            pltpu.VMEM((_EXTRA,), jnp.int32),
            pltpu.VMEM((_EXTRA,), jnp.int32),
            pltpu.VMEM((_EXTRA, 128), jnp.float32),
            pltpu.VMEM((_B, 128), jnp.float32),
            pltpu.VMEM((_B,), jnp.float32),
            pltpu.VMEM_SHARED((16 * _B, 128), jnp.float32),
            pltpu.VMEM_SHARED((16 * _B,), jnp.float32),
            pltpu.SemaphoreType.DMA,
            pltpu.SemaphoreType.DMA,
            pltpu.SemaphoreType.DMA,
            pltpu.SemaphoreType.DMA,
            pltpu.SemaphoreType.DMA,
            pltpu.SemaphoreType.DMA,
        ],
    )
    def gather_kernel(bn_hbm, nbi_hbm, emb_hbm, z2_hbm, z1_hbm,
                      sum_out, cnt_out,
                      idx_v, idxc_v0, idxc_v1, bbi_v0, bbi_v1,
                      bbi2_v0, bbi2_v1, ones_v,
                      rows_v0, rows_v1, xidx_v, xbbi_v, xbbi2_v, xrows_v,
                      osum_v, ocnt_v, acc_sh, cnt_sh,
                      bsem0, bsem1, rsem0, rsem1, xsem1, xsem2):
        c = lax.axis_index("c")
        s = lax.axis_index("s")
        wid = c * 16 + s
        idxc_b = (idxc_v0, idxc_v1)
        bbi_b = (bbi_v0, bbi_v1)
        bbi2_b = (bbi2_v0, bbi2_v1)
        rows_b = (rows_v0, rows_v1)
        bsem_b = (bsem0, bsem1)
        rsem_b = (rsem0, rsem1)

        for o in _lane_offsets(_CH):
            ones_v[pl.ds(o, 16)] = jnp.ones((16,), jnp.float32)

        @pl.when(s == 0)
        def _():
            pltpu.sync_copy(z2_hbm, acc_sh)
            pltpu.sync_copy(z1_hbm, cnt_sh)

        plsc.subcore_barrier()

        pltpu.sync_copy(bn_hbm.at[pl.ds(wid * _PER_TILE, _PER_TILE)], idx_v)

        def start_gathers(k):
            p = k % 2
            for o in _lane_offsets(_CH):
                idxc_b[p][pl.ds(o, 16)] = idx_v[pl.ds(k * _CH + o, 16)]
            cp1 = pltpu.async_copy(nbi_hbm.at[idxc_b[p]], bbi_b[p], bsem_b[p])
            cp2 = pltpu.async_copy(emb_hbm.at[idxc_b[p]], rows_b[p], rsem_b[p])
            return cp1, cp2

        pend = start_gathers(0)
        for k in range(_ROUNDS):
            p = k % 2
            cp1, cp2 = pend
            if k + 1 < _ROUNDS:
                pend = start_gathers(k + 1)
            cp1.wait()
            cp2.wait()
            for o in _lane_offsets(_CH):
                bbi2_b[p][pl.ds(o, 16)] = bbi_b[p][pl.ds(o, 16)] + s * _B
            pltpu.sync_copy(rows_b[p], acc_sh.at[bbi2_b[p]], add=True)
            pltpu.sync_copy(ones_v, cnt_sh.at[bbi2_b[p]], add=True)

        @pl.when(jnp.logical_and(c == 0, s == 0))
        def _():
            pltpu.sync_copy(bn_hbm.at[pl.ds(_NW * _PER_TILE, _EXTRA)], xidx_v)
            cp1 = pltpu.async_copy(nbi_hbm.at[xidx_v], xbbi_v, xsem1)
            cp2 = pltpu.async_copy(emb_hbm.at[xidx_v], xrows_v, xsem2)
            cp1.wait()
            cp2.wait()
            for o in _lane_offsets(_EXTRA):
                xbbi2_v[pl.ds(o, 16)] = xbbi_v[pl.ds(o, 16)]
            pltpu.sync_copy(xrows_v, acc_sh.at[xbbi2_v], add=True)
            pltpu.sync_copy(ones_v.at[pl.ds(0, _EXTRA)], cnt_sh.at[xbbi2_v],
                            add=True)

        plsc.subcore_barrier()
        pltpu.sync_copy(acc_sh.at[pl.ds(s * _B, _B)], osum_v)
        pltpu.sync_copy(cnt_sh.at[pl.ds(s * _B, _B)], ocnt_v)
        pltpu.sync_copy(osum_v, sum_out.at[wid])
        pltpu.sync_copy(ocnt_v, cnt_out.at[pl.ds(wid * _B, _B)])

    return gather_kernel


def _region_body(rbi_ref, reg_ref, accr_ref, cntr_ref):
    i = pl.program_id(0)

    @pl.when(i == 0)
    def _():
        accr_ref[:, :] = jnp.zeros_like(accr_ref)
        cntr_ref[:, :] = jnp.zeros_like(cntr_ref)

    rbi = rbi_ref[0]
    segr = lax.broadcasted_iota(jnp.int32, (_B, _RBLK), 0)
    mr = (segr == rbi).astype(jnp.float32)
    accr_ref[:, :] += jnp.dot(mr, reg_ref[:, :],
                              preferred_element_type=jnp.float32)
    cntr_ref[:, :] += jnp.sum(mr, axis=1, keepdims=True)


def _combine_body(sums_ref, cnts_ref, accr_ref, cntr_ref,
                  ws1_ref, bs1_ref, ws2_ref, bs2_ref, wb1_ref, bb1_ref,
                  wb2_ref, bb2_ref, wv1a_ref, wv1b_ref, bv1_ref,
                  wv2_ref, bv2_ref, out_ref):
    acc_n = sums_ref[pl.ds(0, _B), :]
    for k in range(1, _NW):
        acc_n = acc_n + sums_ref[pl.ds(k * _B, _B), :]
    cnt_row = jnp.sum(cnts_ref[:, :], axis=0, keepdims=True)
    eye = (lax.broadcasted_iota(jnp.int32, (_B, _B), 0)
           == lax.broadcasted_iota(jnp.int32, (_B, _B), 1))
    cnt_n = jnp.sum(jnp.where(eye, cnt_row, 0.0), axis=1, keepdims=True)

    region_mean = accr_ref[:, :] / jnp.maximum(cntr_ref[:, :], 1.0)
    h = jnp.maximum(
        jnp.dot(region_mean, ws1_ref[:, :], preferred_element_type=jnp.float32)
        + bs1_ref[:, :], 0.0)
    gs = jnp.maximum(
        jnp.dot(h, ws2_ref[:, :], preferred_element_type=jnp.float32)
        + bs2_ref[:, :], 0.0)

    bmean = acc_n / jnp.maximum(cnt_n, 1.0)
    hb = jnp.maximum(
        jnp.dot(bmean, wb1_ref[:, :], preferred_element_type=jnp.float32)
        + bb1_ref[:, :], 0.0)
    binfo = (jnp.dot(hb, wb2_ref[:, :], preferred_element_type=jnp.float32)
             + bb2_ref[:, :])

    hv = jnp.maximum(
        jnp.dot(gs, wv1a_ref[:, :], preferred_element_type=jnp.float32)
        + jnp.dot(binfo, wv1b_ref[:, :], preferred_element_type=jnp.float32)
        + bv1_ref[:, :], 0.0)
    out_ref[:, :] = (jnp.dot(hv, wv2_ref[:, :],
                             preferred_element_type=jnp.float32)
                     + bv2_ref[:, :])


def _full_spec(arr):
    nd = arr.ndim
    return pl.BlockSpec(arr.shape, lambda i: (0,) * nd)


def kernel(node_embeddings, region_embeddings, boundary_nodes,
           node_batch_idx, region_batch_idx, action_mask,
           W_s1, b_s1, W_s2, b_s2, W_b1, b_b1, W_b2, b_b2,
           W_v1, b_v1, W_v2, b_v2):
    del action_mask

    sums, cnts = _build_gather()(
        boundary_nodes.astype(jnp.int32),
        node_batch_idx.astype(jnp.int32),
        node_embeddings,
        jnp.zeros((16 * _B, 128), jnp.float32),
        jnp.zeros((16 * _B,), jnp.float32),
    )

    rbi3 = region_batch_idx.astype(jnp.int32).reshape(_NRBLK, 1, _RBLK)
    accr, cntr = pl.pallas_call(
        _region_body,
        grid=(_NRBLK,),
        in_specs=[
            pl.BlockSpec((1, 1, _RBLK), lambda i: (i, 0, 0)),
            pl.BlockSpec((_RBLK, 128), lambda i: (i, 0)),
        ],
        out_specs=[
            pl.BlockSpec((_B, 128), lambda i: (0, 0)),
            pl.BlockSpec((_B, 1), lambda i: (0, 0)),
        ],
        out_shape=[
            jax.ShapeDtypeStruct((_B, 128), jnp.float32),
            jax.ShapeDtypeStruct((_B, 1), jnp.float32),
        ],
    )(rbi3, region_embeddings)

    wv1a = W_v1[:128]
    wv1b = W_v1[128:]
    weights = (W_s1, b_s1.reshape(1, -1), W_s2, b_s2.reshape(1, -1),
               W_b1, b_b1.reshape(1, -1), W_b2, b_b2.reshape(1, -1),
               wv1a, wv1b, b_v1.reshape(1, -1), W_v2, b_v2.reshape(1, -1))

    sums2 = sums.reshape(_NW * _B, 128)
    cnts2 = cnts.reshape(_NW, _B)

    out = pl.pallas_call(
        _combine_body,
        grid=(1,),
        in_specs=[
            pl.BlockSpec((_NW * _B, 128), lambda i: (0, 0)),
            pl.BlockSpec((_NW, _B), lambda i: (0, 0)),
            pl.BlockSpec((_B, 128), lambda i: (0, 0)),
            pl.BlockSpec((_B, 1), lambda i: (0, 0)),
        ] + [_full_spec(w) for w in weights],
        out_specs=pl.BlockSpec((_B, 1), lambda i: (0, 0)),
        out_shape=jax.ShapeDtypeStruct((_B, 1), jnp.float32),
    )(sums2, cnts2, accr, cntr, *weights)
    return out[:, 0]


# speedup vs baseline: 1.5058x; 1.0296x over previous
import functools

import jax
import jax.numpy as jnp
from jax import lax
from jax.experimental import pallas as pl
from jax.experimental.pallas import tpu as pltpu
from jax.experimental.pallas import tpu_sc as plsc

_N_NODES = 100000
_N_REGIONS = 8192
_N_BOUNDARY = 50000
_B = 64

_NW = 32
_PER_TILE = 1560
_CH = 120
_ROUNDS = _PER_TILE // _CH
_EXTRA = _N_BOUNDARY - _NW * _PER_TILE


def _lane_offsets(n):
    offs = list(range(0, n - 15, 16))
    if n % 16:
        offs.append(n - 16)
    return offs

_RBLK = 4096
_NRBLK = _N_REGIONS // _RBLK


@functools.lru_cache(maxsize=1)
def _build_gather():
    mesh = plsc.VectorSubcoreMesh(core_axis_name="c", subcore_axis_name="s")

    @functools.partial(
        pl.kernel,
        mesh=mesh,
        out_type=(
            jax.ShapeDtypeStruct((_NW * _B, 128), jnp.float32),
            jax.ShapeDtypeStruct((_NW * _B,), jnp.float32),
        ),
        scratch_types=[
            pltpu.VMEM((_PER_TILE,), jnp.int32),
            pltpu.VMEM((_CH,), jnp.int32),
            pltpu.VMEM((_CH,), jnp.int32),
            pltpu.VMEM((_CH,), jnp.int32),
            pltpu.VMEM((_CH,), jnp.int32),
            pltpu.VMEM((_CH,), jnp.int32),
            pltpu.VMEM((_CH,), jnp.int32),
            pltpu.VMEM((_CH,), jnp.int32),
            pltpu.VMEM((_CH,), jnp.int32),
            pltpu.VMEM((_CH,), jnp.int32),
            pltpu.VMEM((_CH,), jnp.float32),
            pltpu.VMEM((_CH, 128), jnp.float32),
            pltpu.VMEM((_CH, 128), jnp.float32),
            pltpu.VMEM((_CH, 128), jnp.float32),
            pltpu.VMEM((_EXTRA,), jnp.int32),
            pltpu.VMEM((_EXTRA,), jnp.int32),
            pltpu.VMEM((_EXTRA,), jnp.int32),
            pltpu.VMEM((_EXTRA, 128), jnp.float32),
            pltpu.VMEM((_B, 128), jnp.float32),
            pltpu.VMEM((_B,), jnp.float32),
            pltpu.VMEM_SHARED((16 * _B, 128), jnp.float32),
            pltpu.VMEM_SHARED((16 * _B,), jnp.float32),
            pltpu.SemaphoreType.DMA,
            pltpu.SemaphoreType.DMA,
            pltpu.SemaphoreType.DMA,
            pltpu.SemaphoreType.DMA,
            pltpu.SemaphoreType.DMA,
            pltpu.SemaphoreType.DMA,
            pltpu.SemaphoreType.DMA,
            pltpu.SemaphoreType.DMA,
        ],
    )
    def gather_kernel(bn_hbm, nbi_hbm, emb_hbm, zrow_hbm,
                      sum_out, cnt_out,
                      idx_v, idxc_v0, idxc_v1, idxc_v2,
                      bbi_v0, bbi_v1, bbi_v2,
                      bbi2_v0, bbi2_v1, bbi2_v2, ones_v,
                      rows_v0, rows_v1, rows_v2,
                      xidx_v, xbbi_v, xbbi2_v, xrows_v,
                      osum_v, ocnt_v, acc_sh, cnt_sh,
                      bsem0, bsem1, bsem2, rsem0, rsem1, rsem2, xsem1, xsem2):
        c = lax.axis_index("c")
        s = lax.axis_index("s")
        wid = c * 16 + s
        idxc_b = (idxc_v0, idxc_v1, idxc_v2)
        bbi_b = (bbi_v0, bbi_v1, bbi_v2)
        bbi2_b = (bbi2_v0, bbi2_v1, bbi2_v2)
        rows_b = (rows_v0, rows_v1, rows_v2)
        bsem_b = (bsem0, bsem1, bsem2)
        rsem_b = (rsem0, rsem1, rsem2)

        for o in _lane_offsets(_CH):
            ones_v[pl.ds(o, 16)] = jnp.ones((16,), jnp.float32)

        for o in _lane_offsets(_B):
            ocnt_v[pl.ds(o, 16)] = jnp.zeros((16,), jnp.float32)
        pltpu.sync_copy(zrow_hbm, acc_sh.at[pl.ds(s * _B, _B)])
        pltpu.sync_copy(ocnt_v, cnt_sh.at[pl.ds(s * _B, _B)])

        pltpu.sync_copy(bn_hbm.at[pl.ds(wid * _PER_TILE, _PER_TILE)], idx_v)

        def start_gathers(k):
            p = k % 3
            for o in _lane_offsets(_CH):
                idxc_b[p][pl.ds(o, 16)] = idx_v[pl.ds(k * _CH + o, 16)]
            cp1 = pltpu.async_copy(nbi_hbm.at[idxc_b[p]], bbi_b[p], bsem_b[p])
            cp2 = pltpu.async_copy(emb_hbm.at[idxc_b[p]], rows_b[p], rsem_b[p])
            return cp1, cp2

        pend0 = start_gathers(0)
        pend1 = start_gathers(1)
        pend = (pend0, pend1)
        for k in range(_ROUNDS):
            p = k % 3
            cp1, cp2 = pend[0]
            if k + 2 < _ROUNDS:
                pend = (pend[1], start_gathers(k + 2))
            else:
                pend = (pend[1], None)
            cp1.wait()
            cp2.wait()
            for o in _lane_offsets(_CH):
                bbi2_b[p][pl.ds(o, 16)] = bbi_b[p][pl.ds(o, 16)] + s * _B
            pltpu.sync_copy(rows_b[p], acc_sh.at[bbi2_b[p]], add=True)
            pltpu.sync_copy(ones_v, cnt_sh.at[bbi2_b[p]], add=True)

        @pl.when(jnp.logical_and(c == 0, s == 0))
        def _():
            pltpu.sync_copy(bn_hbm.at[pl.ds(_NW * _PER_TILE, _EXTRA)], xidx_v)
            cp1 = pltpu.async_copy(nbi_hbm.at[xidx_v], xbbi_v, xsem1)
            cp2 = pltpu.async_copy(emb_hbm.at[xidx_v], xrows_v, xsem2)
            cp1.wait()
            cp2.wait()
            for o in _lane_offsets(_EXTRA):
                xbbi2_v[pl.ds(o, 16)] = xbbi_v[pl.ds(o, 16)]
            pltpu.sync_copy(xrows_v, acc_sh.at[xbbi2_v], add=True)
            pltpu.sync_copy(ones_v.at[pl.ds(0, _EXTRA)], cnt_sh.at[xbbi2_v],
                            add=True)

        pltpu.sync_copy(acc_sh.at[pl.ds(s * _B, _B)], osum_v)
        pltpu.sync_copy(cnt_sh.at[pl.ds(s * _B, _B)], ocnt_v)
        pltpu.sync_copy(osum_v, sum_out.at[pl.ds(wid * _B, _B)])
        pltpu.sync_copy(ocnt_v, cnt_out.at[pl.ds(wid * _B, _B)])

    return gather_kernel


def _region_body(rbi_ref, reg_ref, accr_ref, cntr_ref):
    i = pl.program_id(0)

    @pl.when(i == 0)
    def _():
        accr_ref[:, :] = jnp.zeros_like(accr_ref)
        cntr_ref[:, :] = jnp.zeros_like(cntr_ref)

    rbi = rbi_ref[0]
    segr = lax.broadcasted_iota(jnp.int32, (_B, _RBLK), 0)
    mr = (segr == rbi).astype(jnp.float32)
    accr_ref[:, :] += jnp.dot(mr, reg_ref[:, :],
                              preferred_element_type=jnp.float32)
    cntr_ref[:, :] += jnp.sum(mr, axis=1, keepdims=True)


def _combine_body(sums_ref, cnts_ref, accr_ref, cntr_ref,
                  ws1_ref, bs1_ref, ws2_ref, bs2_ref, wb1_ref, bb1_ref,
                  wb2_ref, bb2_ref, wv1_ref, bv1_ref,
                  wv2_ref, bv2_ref, out_ref):
    acc_n = sums_ref[pl.ds(0, _B), :]
    for k in range(1, _NW):
        acc_n = acc_n + sums_ref[pl.ds(k * _B, _B), :]
    colsum = jnp.sum(cnts_ref[:, :], axis=0, keepdims=True)
    cnt_row = colsum[:, :_B] + colsum[:, _B:]
    eye = (lax.broadcasted_iota(jnp.int32, (_B, _B), 0)
           == lax.broadcasted_iota(jnp.int32, (_B, _B), 1))
    cnt_n = jnp.sum(jnp.where(eye, cnt_row, 0.0), axis=1, keepdims=True)

    region_mean = accr_ref[:, :] / jnp.maximum(cntr_ref[:, :], 1.0)
    h = jnp.maximum(
        jnp.dot(region_mean, ws1_ref[:, :], preferred_element_type=jnp.float32)
        + bs1_ref[:, :], 0.0)
    gs = jnp.maximum(
        jnp.dot(h, ws2_ref[:, :], preferred_element_type=jnp.float32)
        + bs2_ref[:, :], 0.0)

    bmean = acc_n / jnp.maximum(cnt_n, 1.0)
    hb = jnp.maximum(
        jnp.dot(bmean, wb1_ref[:, :], preferred_element_type=jnp.float32)
        + bb1_ref[:, :], 0.0)
    binfo = (jnp.dot(hb, wb2_ref[:, :], preferred_element_type=jnp.float32)
             + bb2_ref[:, :])

    hv = jnp.maximum(
        jnp.dot(gs, wv1_ref[pl.ds(0, 128), :],
                preferred_element_type=jnp.float32)
        + jnp.dot(binfo, wv1_ref[pl.ds(128, _B), :],
                  preferred_element_type=jnp.float32)
        + bv1_ref[:, :], 0.0)
    out_ref[:, :] = (jnp.dot(hv, wv2_ref[:, :],
                             preferred_element_type=jnp.float32)
                     + bv2_ref[:, :])


def _full_spec(arr):
    nd = arr.ndim
    return pl.BlockSpec(arr.shape, lambda i: (0,) * nd)


def kernel(node_embeddings, region_embeddings, boundary_nodes,
           node_batch_idx, region_batch_idx, action_mask,
           W_s1, b_s1, W_s2, b_s2, W_b1, b_b1, W_b2, b_b2,
           W_v1, b_v1, W_v2, b_v2):
    del action_mask

    sums, cnts = _build_gather()(
        boundary_nodes.astype(jnp.int32),
        node_batch_idx.astype(jnp.int32),
        node_embeddings,
        jnp.zeros((_B, 128), jnp.float32),
    )

    rbi3 = region_batch_idx.astype(jnp.int32).reshape(_NRBLK, 1, _RBLK)
    accr, cntr = pl.pallas_call(
        _region_body,
        grid=(_NRBLK,),
        in_specs=[
            pl.BlockSpec((1, 1, _RBLK), lambda i: (i, 0, 0)),
            pl.BlockSpec((_RBLK, 128), lambda i: (i, 0)),
        ],
        out_specs=[
            pl.BlockSpec((_B, 128), lambda i: (0, 0)),
            pl.BlockSpec((_B, 1), lambda i: (0, 0)),
        ],
        out_shape=[
            jax.ShapeDtypeStruct((_B, 128), jnp.float32),
            jax.ShapeDtypeStruct((_B, 1), jnp.float32),
        ],
    )(rbi3, region_embeddings)

    weights = (W_s1, b_s1.reshape(1, -1), W_s2, b_s2.reshape(1, -1),
               W_b1, b_b1.reshape(1, -1), W_b2, b_b2.reshape(1, -1),
               W_v1, b_v1.reshape(1, -1), W_v2, b_v2.reshape(1, -1))

    cnts2 = cnts.reshape(_NW * _B // 128, 128)

    out = pl.pallas_call(
        _combine_body,
        grid=(1,),
        in_specs=[
            pl.BlockSpec((_NW * _B, 128), lambda i: (0, 0)),
            pl.BlockSpec((_NW * _B // 128, 128), lambda i: (0, 0)),
            pl.BlockSpec((_B, 128), lambda i: (0, 0)),
            pl.BlockSpec((_B, 1), lambda i: (0, 0)),
        ] + [_full_spec(w) for w in weights],
        out_specs=pl.BlockSpec((_B, 1), lambda i: (0, 0)),
        out_shape=jax.ShapeDtypeStruct((_B, 1), jnp.float32),
    )(sums, cnts2, accr, cntr, *weights)
    return out[:, 0]


# speedup vs baseline: 1.5685x; 1.0417x over previous
import functools

import jax
import jax.numpy as jnp
from jax import lax
from jax.experimental import pallas as pl
from jax.experimental.pallas import tpu as pltpu
from jax.experimental.pallas import tpu_sc as plsc

_N_NODES = 100000
_N_REGIONS = 8192
_N_BOUNDARY = 50000
_B = 64

_NW = 32
_PER_TILE = 1560
_CH = 120
_ROUNDS = _PER_TILE // _CH
_EXTRA = _N_BOUNDARY - _NW * _PER_TILE


def _lane_offsets(n):
    offs = list(range(0, n - 15, 16))
    if n % 16:
        offs.append(n - 16)
    return offs

_RBLK = 4096
_NRBLK = _N_REGIONS // _RBLK


@functools.lru_cache(maxsize=1)
def _build_gather():
    mesh = plsc.VectorSubcoreMesh(core_axis_name="c", subcore_axis_name="s")

    @functools.partial(
        pl.kernel,
        mesh=mesh,
        out_type=(
            jax.ShapeDtypeStruct((_NW * _B, 128), jnp.float32),
            jax.ShapeDtypeStruct((_NW * _B,), jnp.float32),
        ),
        scratch_types=[
            pltpu.VMEM((_PER_TILE,), jnp.int32),
            pltpu.VMEM((_CH,), jnp.int32),
            pltpu.VMEM((_CH,), jnp.int32),
            pltpu.VMEM((_CH,), jnp.int32),
            pltpu.VMEM((_CH,), jnp.int32),
            pltpu.VMEM((_CH,), jnp.int32),
            pltpu.VMEM((_CH,), jnp.int32),
            pltpu.VMEM((_CH,), jnp.float32),
            pltpu.VMEM((_CH, 128), jnp.float32),
            pltpu.VMEM((_CH, 128), jnp.float32),
            pltpu.VMEM((_CH, 128), jnp.float32),
            pltpu.VMEM((_EXTRA,), jnp.int32),
            pltpu.VMEM((_EXTRA,), jnp.int32),
            pltpu.VMEM((_EXTRA,), jnp.int32),
            pltpu.VMEM((_EXTRA, 128), jnp.float32),
            pltpu.VMEM((_B, 128), jnp.float32),
            pltpu.VMEM((_B,), jnp.float32),
            pltpu.VMEM_SHARED((16 * _B, 128), jnp.float32),
            pltpu.VMEM_SHARED((16 * _B,), jnp.float32),
            pltpu.SemaphoreType.DMA,
            pltpu.SemaphoreType.DMA,
            pltpu.SemaphoreType.DMA,
            pltpu.SemaphoreType.DMA,
            pltpu.SemaphoreType.DMA,
            pltpu.SemaphoreType.DMA,
            pltpu.SemaphoreType.DMA,
            pltpu.SemaphoreType.DMA,
        ],
    )
    def gather_kernel(bn_hbm, nbi_hbm, emb_hbm, zrow_hbm,
                      sum_out, cnt_out,
                      idx_v,
                      bbi_v0, bbi_v1, bbi_v2,
                      bbi2_v0, bbi2_v1, bbi2_v2, ones_v,
                      rows_v0, rows_v1, rows_v2,
                      xidx_v, xbbi_v, xbbi2_v, xrows_v,
                      osum_v, ocnt_v, acc_sh, cnt_sh,
                      bsem0, bsem1, bsem2, rsem0, rsem1, rsem2, xsem1, xsem2):
        c = lax.axis_index("c")
        s = lax.axis_index("s")
        wid = c * 16 + s
        bbi_b = (bbi_v0, bbi_v1, bbi_v2)
        bbi2_b = (bbi2_v0, bbi2_v1, bbi2_v2)
        rows_b = (rows_v0, rows_v1, rows_v2)
        bsem_b = (bsem0, bsem1, bsem2)
        rsem_b = (rsem0, rsem1, rsem2)

        for o in _lane_offsets(_CH):
            ones_v[pl.ds(o, 16)] = jnp.ones((16,), jnp.float32)

        for o in _lane_offsets(_B):
            ocnt_v[pl.ds(o, 16)] = jnp.zeros((16,), jnp.float32)
        pltpu.sync_copy(zrow_hbm, acc_sh.at[pl.ds(s * _B, _B)])
        pltpu.sync_copy(ocnt_v, cnt_sh.at[pl.ds(s * _B, _B)])

        pltpu.sync_copy(bn_hbm.at[pl.ds(wid * _PER_TILE, _PER_TILE)], idx_v)

        def start_gathers(k):
            p = k % 3
            sl = idx_v.at[pl.ds(k * _CH, _CH)]
            cp1 = pltpu.async_copy(nbi_hbm.at[sl], bbi_b[p], bsem_b[p])
            cp2 = pltpu.async_copy(emb_hbm.at[sl], rows_b[p], rsem_b[p])
            return cp1, cp2

        pend0 = start_gathers(0)
        pend1 = start_gathers(1)
        pend = (pend0, pend1)
        for k in range(_ROUNDS):
            p = k % 3
            cp1, cp2 = pend[0]
            if k + 2 < _ROUNDS:
                pend = (pend[1], start_gathers(k + 2))
            else:
                pend = (pend[1], None)
            cp1.wait()
            cp2.wait()
            for o in _lane_offsets(_CH):
                bbi2_b[p][pl.ds(o, 16)] = bbi_b[p][pl.ds(o, 16)] + s * _B
            pltpu.sync_copy(rows_b[p], acc_sh.at[bbi2_b[p]], add=True)
            pltpu.sync_copy(ones_v, cnt_sh.at[bbi2_b[p]], add=True)

        @pl.when(jnp.logical_and(c == 0, s == 0))
        def _():
            pltpu.sync_copy(bn_hbm.at[pl.ds(_NW * _PER_TILE, _EXTRA)], xidx_v)
            cp1 = pltpu.async_copy(nbi_hbm.at[xidx_v], xbbi_v, xsem1)
            cp2 = pltpu.async_copy(emb_hbm.at[xidx_v], xrows_v, xsem2)
            cp1.wait()
            cp2.wait()
            for o in _lane_offsets(_EXTRA):
                xbbi2_v[pl.ds(o, 16)] = xbbi_v[pl.ds(o, 16)]
            pltpu.sync_copy(xrows_v, acc_sh.at[xbbi2_v], add=True)
            pltpu.sync_copy(ones_v.at[pl.ds(0, _EXTRA)], cnt_sh.at[xbbi2_v],
                            add=True)

        pltpu.sync_copy(acc_sh.at[pl.ds(s * _B, _B)], osum_v)
        pltpu.sync_copy(cnt_sh.at[pl.ds(s * _B, _B)], ocnt_v)
        pltpu.sync_copy(osum_v, sum_out.at[pl.ds(wid * _B, _B)])
        pltpu.sync_copy(ocnt_v, cnt_out.at[pl.ds(wid * _B, _B)])

    return gather_kernel


def _region_body(rbi_ref, reg_ref, accr_ref, cntr_ref):
    i = pl.program_id(0)

    @pl.when(i == 0)
    def _():
        accr_ref[:, :] = jnp.zeros_like(accr_ref)
        cntr_ref[:, :] = jnp.zeros_like(cntr_ref)

    rbi = rbi_ref[0]
    segr = lax.broadcasted_iota(jnp.int32, (_B, _RBLK), 0)
    mr = (segr == rbi).astype(jnp.float32)
    accr_ref[:, :] += jnp.dot(mr, reg_ref[:, :],
                              preferred_element_type=jnp.float32)
    cntr_ref[:, :] += jnp.sum(mr, axis=1, keepdims=True)


def _combine_body(sums_ref, cnts_ref, accr_ref, cntr_ref,
                  ws1_ref, bs1_ref, ws2_ref, bs2_ref, wb1_ref, bb1_ref,
                  wb2_ref, bb2_ref, wv1_ref, bv1_ref,
                  wv2_ref, bv2_ref, out_ref):
    acc_n = sums_ref[pl.ds(0, _B), :]
    for k in range(1, _NW):
        acc_n = acc_n + sums_ref[pl.ds(k * _B, _B), :]
    colsum = jnp.sum(cnts_ref[:, :], axis=0, keepdims=True)
    cnt_row = colsum[:, :_B] + colsum[:, _B:]
    eye = (lax.broadcasted_iota(jnp.int32, (_B, _B), 0)
           == lax.broadcasted_iota(jnp.int32, (_B, _B), 1))
    cnt_n = jnp.sum(jnp.where(eye, cnt_row, 0.0), axis=1, keepdims=True)

    region_mean = accr_ref[:, :] / jnp.maximum(cntr_ref[:, :], 1.0)
    h = jnp.maximum(
        jnp.dot(region_mean, ws1_ref[:, :], preferred_element_type=jnp.float32)
        + bs1_ref[:, :], 0.0)
    gs = jnp.maximum(
        jnp.dot(h, ws2_ref[:, :], preferred_element_type=jnp.float32)
        + bs2_ref[:, :], 0.0)

    bmean = acc_n / jnp.maximum(cnt_n, 1.0)
    hb = jnp.maximum(
        jnp.dot(bmean, wb1_ref[:, :], preferred_element_type=jnp.float32)
        + bb1_ref[:, :], 0.0)
    binfo = (jnp.dot(hb, wb2_ref[:, :], preferred_element_type=jnp.float32)
             + bb2_ref[:, :])

    hv = jnp.maximum(
        jnp.dot(gs, wv1_ref[pl.ds(0, 128), :],
                preferred_element_type=jnp.float32)
        + jnp.dot(binfo, wv1_ref[pl.ds(128, _B), :],
                  preferred_element_type=jnp.float32)
        + bv1_ref[:, :], 0.0)
    value_col = (jnp.dot(hv, wv2_ref[:, :],
                         preferred_element_type=jnp.float32)
                 + bv2_ref[:, :])
    eye2 = (lax.broadcasted_iota(jnp.int32, (_B, _B), 0)
            == lax.broadcasted_iota(jnp.int32, (_B, _B), 1))
    out_ref[:, :] = jnp.sum(jnp.where(eye2, value_col, 0.0),
                            axis=0, keepdims=True)


def _full_spec(arr):
    nd = arr.ndim
    return pl.BlockSpec(arr.shape, lambda i: (0,) * nd)


def kernel(node_embeddings, region_embeddings, boundary_nodes,
           node_batch_idx, region_batch_idx, action_mask,
           W_s1, b_s1, W_s2, b_s2, W_b1, b_b1, W_b2, b_b2,
           W_v1, b_v1, W_v2, b_v2):
    del action_mask

    sums, cnts = _build_gather()(
        boundary_nodes.astype(jnp.int32),
        node_batch_idx.astype(jnp.int32),
        node_embeddings,
        jnp.zeros((_B, 128), jnp.float32),
    )

    rbi3 = region_batch_idx.astype(jnp.int32).reshape(_NRBLK, 1, _RBLK)
    accr, cntr = pl.pallas_call(
        _region_body,
        grid=(_NRBLK,),
        in_specs=[
            pl.BlockSpec((1, 1, _RBLK), lambda i: (i, 0, 0)),
            pl.BlockSpec((_RBLK, 128), lambda i: (i, 0)),
        ],
        out_specs=[
            pl.BlockSpec((_B, 128), lambda i: (0, 0)),
            pl.BlockSpec((_B, 1), lambda i: (0, 0)),
        ],
        out_shape=[
            jax.ShapeDtypeStruct((_B, 128), jnp.float32),
            jax.ShapeDtypeStruct((_B, 1), jnp.float32),
        ],
    )(rbi3, region_embeddings)

    weights = (W_s1, b_s1.reshape(1, -1), W_s2, b_s2.reshape(1, -1),
               W_b1, b_b1.reshape(1, -1), W_b2, b_b2.reshape(1, -1),
               W_v1, b_v1.reshape(1, -1), W_v2, b_v2.reshape(1, -1))

    cnts2 = cnts.reshape(_NW * _B // 128, 128)

    out = pl.pallas_call(
        _combine_body,
        grid=(1,),
        in_specs=[
            pl.BlockSpec((_NW * _B, 128), lambda i: (0, 0)),
            pl.BlockSpec((_NW * _B // 128, 128), lambda i: (0, 0)),
            pl.BlockSpec((_B, 128), lambda i: (0, 0)),
            pl.BlockSpec((_B, 1), lambda i: (0, 0)),
        ] + [_full_spec(w) for w in weights],
        out_specs=pl.BlockSpec((1, _B), lambda i: (0, 0)),
        out_shape=jax.ShapeDtypeStruct((1, _B), jnp.float32),
    )(sums, cnts2, accr, cntr, *weights)
    return out.reshape(_B)
